# baseline (speedup 1.0000x reference)
import functools

import jax
import jax.numpy as jnp
from jax import lax
from jax.experimental import pallas as pl
from jax.experimental.pallas import tpu as pltpu
from jax.experimental.pallas import tpu_sc as plsc

_N = 4096
_K = 32
_LANES = 16
_NC, _NS = 2, 16
_NW = _NC * _NS
_CHUNK = _N * _K // _NW

_ROWS = 512


def _sc_gather_body(t_hbm, idx_hbm, out_hbm, table_v, idx_v, out_v, sem_t, sem_i):
    wid = lax.axis_index("s") * _NC + lax.axis_index("c")
    base = wid * _CHUNK
    cp_t = pltpu.async_copy(t_hbm, table_v, sem_t)
    cp_i = pltpu.async_copy(idx_hbm.at[pl.ds(base, _CHUNK)], idx_v, sem_i)
    cp_t.wait()
    cp_i.wait()

    @pl.loop(0, _CHUNK // _LANES)
    def _(i):
        iv = idx_v[pl.ds(i * _LANES, _LANES)]
        valid = iv >= jnp.zeros((_LANES,), jnp.int32)
        safe = jnp.maximum(iv, jnp.zeros((_LANES,), jnp.int32))
        g = plsc.load_gather(table_v, [safe])
        out_v[pl.ds(i * _LANES, _LANES)] = jnp.where(
            valid, g, jnp.zeros((_LANES,), jnp.float32))

    pltpu.sync_copy(out_v, out_hbm.at[pl.ds(base, _CHUNK)])


@functools.cache
def _sc_gather():
    return pl.kernel(
        _sc_gather_body,
        out_type=jax.ShapeDtypeStruct((_N * _K,), jnp.float32),
        mesh=plsc.VectorSubcoreMesh(
            core_axis_name="c", subcore_axis_name="s",
            num_cores=_NC, num_subcores=_NS),
        scratch_types=[
            pltpu.VMEM((_N,), jnp.float32),
            pltpu.VMEM((_CHUNK,), jnp.int32),
            pltpu.VMEM((_CHUNK,), jnp.float32),
            pltpu.SemaphoreType.DMA,
            pltpu.SemaphoreType.DMA,
        ],
        compiler_params=pltpu.CompilerParams(needs_layout_passes=False),
    )


def _tc_outer_body(t_ref, n_ref, coef_ref, tgt_ref):
    nb = n_ref[...]
    a3 = jnp.broadcast_to(nb[:, :, None], (_ROWS, _K, _K))
    b3 = jnp.broadcast_to(nb[:, None, :], (_ROWS, _K, _K))
    coef_ref[...] = a3 * b3
    tgt_ref[...] = t_ref[...] * nb


_tc_outer = pl.pallas_call(
    _tc_outer_body,
    grid=(_N // _ROWS,),
    in_specs=[
        pl.BlockSpec((_ROWS, 1), lambda i: (i, 0)),
        pl.BlockSpec((_ROWS, _K), lambda i: (i, 0)),
    ],
    out_specs=[
        pl.BlockSpec((_ROWS, _K, _K), lambda i: (i, 0, 0)),
        pl.BlockSpec((_ROWS, _K), lambda i: (i, 0)),
    ],
    out_shape=[
        jax.ShapeDtypeStruct((_N, _K, _K), jnp.float32),
        jax.ShapeDtypeStruct((_N, _K), jnp.float32),
    ],
)


def kernel(tensor, mask_idxs):
    t_flat = tensor.reshape(-1)
    idx_flat = mask_idxs.astype(jnp.int32).reshape(-1)
    neigh = _sc_gather()(t_flat, idx_flat).reshape(_N, _K)
    coef, target = _tc_outer(t_flat.reshape(_N, 1), neigh)
    return coef, target


# speedup vs baseline: 2.5463x; 2.5463x over previous
import functools

import jax
import jax.numpy as jnp
from jax import lax
from jax.experimental import pallas as pl
from jax.experimental.pallas import tpu as pltpu
from jax.experimental.pallas import tpu_sc as plsc

_N = 4096
_K = 32
_LANES = 16
_NC, _NS = 2, 16
_NW = _NC * _NS
_CHUNK = _N * _K // _NW

_CL = 512


def _sc_gather_body(t_hbm, idx_hbm, out_hbm, table_v, idx_v, out_v,
                    sem_t, sem_i):
    w = lax.axis_index("s") * _NC + lax.axis_index("c")
    base = w * _CHUNK
    cp_t = pltpu.async_copy(t_hbm, table_v, sem_t)
    cp_i = pltpu.async_copy(idx_hbm.at[pl.ds(base, _CHUNK)], idx_v, sem_i)
    cp_t.wait()
    cp_i.wait()
    lane32 = lax.iota(jnp.int32, _LANES) * _K
    zi = jnp.zeros((_LANES,), jnp.int32)
    zf = jnp.zeros((_LANES,), jnp.float32)

    @pl.loop(0, 128 // _LANES)
    def _(j):
        @pl.loop(0, _K)
        def _(k):
            addr = lane32 + (j * (_LANES * _K) + k)
            iv = plsc.load_gather(idx_v, [addr])
            safe = jnp.maximum(iv, zi)
            g = plsc.load_gather(table_v, [safe])
            out_v[pl.ds(k * 128 + j * _LANES, _LANES)] = jnp.where(
                iv >= zi, g, zf)

    pltpu.sync_copy(out_v, out_hbm.at[pl.ds(base, _CHUNK)])


@functools.cache
def _sc_gather():
    return pl.kernel(
        _sc_gather_body,
        out_type=jax.ShapeDtypeStruct((_N * _K,), jnp.float32),
        mesh=plsc.VectorSubcoreMesh(
            core_axis_name="c", subcore_axis_name="s",
            num_cores=_NC, num_subcores=_NS),
        scratch_types=[
            pltpu.VMEM((_N,), jnp.float32),
            pltpu.VMEM((_CHUNK,), jnp.int32),
            pltpu.VMEM((_CHUNK,), jnp.float32),
            pltpu.SemaphoreType.DMA,
            pltpu.SemaphoreType.DMA,
        ],
        compiler_params=pltpu.CompilerParams(needs_layout_passes=False),
    )


def _tc_outer_body(t_ref, n_ref, coef_ref, tgt_ref):
    i = pl.program_id(0)
    for q in range(_CL // 128):
        nbT = n_ref[pl.ds(q * _K, _K), :]
        a3 = jnp.broadcast_to(nbT[:, None, :], (_K, _K, 128))
        b3 = jnp.broadcast_to(nbT[None, :, :], (_K, _K, 128))
        coef_ref[:, :, pl.ds(q * 128, 128)] = a3 * b3
        tb = t_ref[pl.ds(i * (_CL // 128) + q, 1), :]
        tgt_ref[:, pl.ds(q * 128, 128)] = jnp.broadcast_to(tb, (_K, 128)) * nbT


_tc_outer = pl.pallas_call(
    _tc_outer_body,
    grid=(_N // _CL,),
    in_specs=[
        pl.BlockSpec((_N // 128, 128), lambda i: (0, 0)),
        pl.BlockSpec((_CL // 128 * _K, 128), lambda i: (i, 0)),
    ],
    out_specs=[
        pl.BlockSpec((_K, _K, _CL), lambda i: (0, 0, i)),
        pl.BlockSpec((_K, _CL), lambda i: (0, i)),
    ],
    out_shape=[
        jax.ShapeDtypeStruct((_K, _K, _N), jnp.float32),
        jax.ShapeDtypeStruct((_K, _N), jnp.float32),
    ],
)


def kernel(tensor, mask_idxs):
    t_flat = tensor.reshape(-1)
    idx_flat = mask_idxs.astype(jnp.int32).reshape(-1)
    neighT = _sc_gather()(t_flat, idx_flat)
    coefT, tgtT = _tc_outer(
        t_flat.reshape(_N // 128, 128),
        neighT.reshape(_N // 128 * _K, 128))
    return jnp.transpose(coefT, (2, 0, 1)), tgtT.T
